# Initial kernel scaffold; baseline (speedup 1.0000x reference)
import jax
import jax.numpy as jnp
from jax.experimental import pallas as pl


def kernel(x, edge_index, edge_weight, W1, b1, W2, b2, W3, b3):
    raise NotImplementedError("write your pallas kernel here")


# speedup vs baseline: 3.7412x; 3.7412x over previous
import functools

import jax
import jax.numpy as jnp
from jax import lax
from jax.experimental import pallas as pl
from jax.experimental.pallas import tpu as pltpu
from jax.experimental.pallas import tpu_sc as plsc

NC = 2
NS = 16
L = 16
NW = NC * NS


def _make_spmm(N, E, D, K=80):
    EW = E // NW
    assert EW * NW == E and EW % K == 0 and K % 8 == 0 and K <= 128
    nchunk = EW // K
    rows_a = ((N + NS - 1) // NS + 7) // 8 * 8
    rows_b = N - rows_a * (NS - 1)
    assert rows_b > 0
    mesh = plsc.VectorSubcoreMesh(core_axis_name="c", subcore_axis_name="s")

    @functools.partial(
        pl.kernel,
        out_type=jax.ShapeDtypeStruct((NC, N, D), jnp.float32),
        mesh=mesh,
        compiler_params=pltpu.CompilerParams(needs_layout_passes=False,
                                             use_tc_tiling_on_sc=False),
        scratch_types=[
            pltpu.VMEM((K,), jnp.int32),
            pltpu.VMEM((K,), jnp.int32),
            pltpu.VMEM((K,), jnp.float32),
            pltpu.VMEM((K, D), jnp.float32),
            pltpu.VMEM_SHARED((N, D), jnp.float32),
            pltpu.SemaphoreType.DMA,
        ],
    )
    def spmm(src_hbm, dst_hbm, w_hbm, table_hbm, zeros_hbm, out_hbm,
             src_v, dst_v, w_v, rows_v, acc_sh, sem):
        c = lax.axis_index("c")
        s = lax.axis_index("s")
        wid = s * NC + c

        @pl.when(s < NS - 1)
        def _():
            base = pl.multiple_of(s * rows_a, 8)
            pltpu.sync_copy(zeros_hbm.at[pl.ds(base, rows_a)],
                            acc_sh.at[pl.ds(base, rows_a)])

        @pl.when(s == NS - 1)
        def _():
            pltpu.sync_copy(zeros_hbm.at[pl.ds((NS - 1) * rows_a, rows_b)],
                            acc_sh.at[pl.ds((NS - 1) * rows_a, rows_b)])

        plsc.subcore_barrier()

        ebase = wid * EW

        def chunk(g, carry):
            off = pl.multiple_of(ebase + g * K, 8)
            pltpu.sync_copy(src_hbm.at[pl.ds(off, K)], src_v)
            pltpu.sync_copy(dst_hbm.at[pl.ds(off, K)], dst_v)
            pltpu.sync_copy(w_hbm.at[pl.ds(off, K)], w_v)
            pltpu.async_copy(table_hbm.at[src_v], rows_v, sem).wait()

            def edge(e, cin):
                wb = plsc.load_gather(w_v, [jnp.full((L,), e, jnp.int32)])
                for j in range(D // L):
                    sl = pl.ds(j * L, L)
                    rows_v[e, sl] = rows_v[e, sl] * wb
                return cin

            lax.fori_loop(0, K, edge, 0)
            pltpu.sync_copy(rows_v, acc_sh.at[dst_v], add=True)
            return carry

        lax.fori_loop(0, nchunk, chunk, 0)
        plsc.subcore_barrier()

        @pl.when(s < NS - 1)
        def _():
            base = pl.multiple_of(s * rows_a, 8)
            pltpu.sync_copy(acc_sh.at[pl.ds(base, rows_a)],
                            out_hbm.at[c, pl.ds(base, rows_a)])

        @pl.when(s == NS - 1)
        def _():
            pltpu.sync_copy(acc_sh.at[pl.ds((NS - 1) * rows_a, rows_b)],
                            out_hbm.at[c, pl.ds((NS - 1) * rows_a, rows_b)])

    return spmm


def _matmul(x, W, bm=1000):
    n, f = x.shape
    h = W.shape[1]
    grid = n // bm

    def body(x_ref, w_ref, o_ref):
        o_ref[...] = jnp.dot(x_ref[...], w_ref[...],
                             preferred_element_type=jnp.float32)

    return pl.pallas_call(
        body,
        grid=(grid,),
        in_specs=[pl.BlockSpec((bm, f), lambda i: (i, 0)),
                  pl.BlockSpec((f, h), lambda i: (0, 0))],
        out_specs=pl.BlockSpec((bm, h), lambda i: (i, 0)),
        out_shape=jax.ShapeDtypeStruct((n, h), jnp.float32),
    )(x, W)


def _fuse_matmul(p, b, W, relu, bm=1000):
    _, n, d = p.shape
    h = W.shape[1]
    grid = n // bm
    b2 = b.reshape(1, d)

    def body(p_ref, b_ref, w_ref, o_ref):
        z = p_ref[0] + p_ref[1] + b_ref[...]
        if relu:
            z = jnp.maximum(z, 0.0)
        o_ref[...] = jnp.dot(z, w_ref[...], preferred_element_type=jnp.float32)

    return pl.pallas_call(
        body,
        grid=(grid,),
        in_specs=[pl.BlockSpec((2, bm, d), lambda i: (0, i, 0)),
                  pl.BlockSpec((1, d), lambda i: (0, 0)),
                  pl.BlockSpec((d, h), lambda i: (0, 0))],
        out_specs=pl.BlockSpec((bm, h), lambda i: (i, 0)),
        out_shape=jax.ShapeDtypeStruct((n, h), jnp.float32),
    )(p, b2, W)


def _fuse_logsoftmax(p, b, bm=1000):
    _, n, d = p.shape
    grid = n // bm
    b2 = b.reshape(1, d)

    def body(p_ref, b_ref, o_ref):
        z = p_ref[0] + p_ref[1] + b_ref[...]
        z = z - jnp.max(z, axis=1, keepdims=True)
        o_ref[...] = z - jnp.log(jnp.sum(jnp.exp(z), axis=1, keepdims=True))

    return pl.pallas_call(
        body,
        grid=(grid,),
        in_specs=[pl.BlockSpec((2, bm, d), lambda i: (0, i, 0)),
                  pl.BlockSpec((1, d), lambda i: (0, 0))],
        out_specs=pl.BlockSpec((bm, d), lambda i: (i, 0)),
        out_shape=jax.ShapeDtypeStruct((n, d), jnp.float32),
    )(p, b2)


def kernel(x, edge_index, edge_weight, W1, b1, W2, b2, W3, b3):
    n, nfeat = x.shape
    e = edge_weight.shape[0]
    nhid = W1.shape[1]
    nclass = W3.shape[1]
    src = edge_index[0]
    dst = edge_index[1]
    zeros_h = jnp.zeros((n, nhid), jnp.float32)
    zeros_c = jnp.zeros((n, nclass), jnp.float32)

    spmm_h = _make_spmm(n, e, nhid)
    spmm_c = _make_spmm(n, e, nclass)

    s1 = _matmul(x, W1)
    p1 = spmm_h(src, dst, edge_weight, s1, zeros_h)
    s2 = _fuse_matmul(p1, b1, W2, relu=True)
    p2 = spmm_h(src, dst, edge_weight, s2, zeros_h)
    s3 = _fuse_matmul(p2, b2, W3, relu=False)
    p3 = spmm_c(src, dst, edge_weight, s3, zeros_c)
    return _fuse_logsoftmax(p3, b3)


# speedup vs baseline: 9.5963x; 2.5650x over previous
import functools

import jax
import jax.numpy as jnp
from jax import lax
from jax.experimental import pallas as pl
from jax.experimental.pallas import tpu as pltpu
from jax.experimental.pallas import tpu_sc as plsc

NC = 2
NS = 16
L = 16
NW = NC * NS


def _make_spmm(N, E, D, K=80):
    EW = E // NW
    assert EW * NW == E and EW % K == 0 and K % 8 == 0 and K <= 128
    nchunk = EW // K
    assert nchunk % 2 == 1
    npair = nchunk // 2
    rows_a = ((N + NS - 1) // NS + 7) // 8 * 8
    rows_b = N - rows_a * (NS - 1)
    assert rows_b > 0
    mesh = plsc.VectorSubcoreMesh(core_axis_name="c", subcore_axis_name="s")

    @functools.partial(
        pl.kernel,
        out_type=jax.ShapeDtypeStruct((NC, N, D), jnp.float32),
        mesh=mesh,
        compiler_params=pltpu.CompilerParams(needs_layout_passes=False,
                                             use_tc_tiling_on_sc=False),
        scratch_types=[
            pltpu.VMEM((nchunk, K), jnp.int32),
            pltpu.VMEM((nchunk, K), jnp.int32),
            pltpu.VMEM((EW,), jnp.float32),
            pltpu.VMEM((K, D), jnp.float32),
            pltpu.VMEM((K, D), jnp.float32),
            pltpu.VMEM_SHARED((N, D), jnp.float32),
            pltpu.SemaphoreType.DMA,
            pltpu.SemaphoreType.DMA,
            pltpu.SemaphoreType.DMA,
        ],
    )
    def spmm(src_hbm, dst_hbm, w_hbm, table_hbm, zeros_hbm, out_hbm,
             src_v, dst_v, w_v, rows_a_v, rows_b_v, acc_sh,
             gsem_a, gsem_b, ssem_b):
        c = lax.axis_index("c")
        s = lax.axis_index("s")
        wid = s * NC + c

        pltpu.sync_copy(src_hbm.at[pl.ds(wid * nchunk, nchunk)], src_v)
        pltpu.sync_copy(dst_hbm.at[pl.ds(wid * nchunk, nchunk)], dst_v)
        pltpu.sync_copy(w_hbm.at[pl.ds(pl.multiple_of(wid * EW, 8), EW)], w_v)

        @pl.when(s < NS - 1)
        def _():
            base = pl.multiple_of(s * rows_a, 8)
            pltpu.sync_copy(zeros_hbm.at[pl.ds(base, rows_a)],
                            acc_sh.at[pl.ds(base, rows_a)])

        @pl.when(s == NS - 1)
        def _():
            pltpu.sync_copy(zeros_hbm.at[pl.ds((NS - 1) * rows_a, rows_b)],
                            acc_sh.at[pl.ds((NS - 1) * rows_a, rows_b)])

        plsc.subcore_barrier()

        def weight(rows_ref, chunk_id):
            def edge(e, cin):
                wb = plsc.load_gather(
                    w_v, [jnp.full((L,), chunk_id * K + e, jnp.int32)])
                for j in range(D // L):
                    sl = pl.ds(j * L, L)
                    rows_ref[e, sl] = rows_ref[e, sl] * wb
                return cin
            lax.fori_loop(0, K, edge, 0, unroll=8)

        def issue_gather(chunk_id, rows_ref, sem):
            pltpu.async_copy(table_hbm.at[src_v.at[chunk_id]], rows_ref, sem)

        def wait_gather(chunk_id, rows_ref, sem):
            pltpu.make_async_copy(table_hbm.at[src_v.at[chunk_id]],
                                  rows_ref, sem).wait()

        def sync_scatter(chunk_id, rows_ref):
            pltpu.sync_copy(rows_ref, acc_sh.at[dst_v.at[chunk_id]], add=True)

        issue_gather(0, rows_a_v, gsem_a)

        def pair(i, carry):
            a = 2 * i

            @pl.when(i > 0)
            def _():
                pltpu.make_async_copy(rows_b_v, acc_sh.at[dst_v.at[a - 1]],
                                      ssem_b).wait()

            issue_gather(a + 1, rows_b_v, gsem_b)
            wait_gather(a, rows_a_v, gsem_a)
            weight(rows_a_v, a)
            sync_scatter(a, rows_a_v)
            issue_gather(a + 2, rows_a_v, gsem_a)
            wait_gather(a + 1, rows_b_v, gsem_b)
            weight(rows_b_v, a + 1)
            pltpu.async_copy(rows_b_v, acc_sh.at[dst_v.at[a + 1]],
                             ssem_b, add=True)
            return carry

        lax.fori_loop(0, npair, pair, 0)
        pltpu.make_async_copy(rows_b_v, acc_sh.at[dst_v.at[nchunk - 2]],
                              ssem_b).wait()
        wait_gather(nchunk - 1, rows_a_v, gsem_a)
        weight(rows_a_v, nchunk - 1)
        sync_scatter(nchunk - 1, rows_a_v)
        plsc.subcore_barrier()

        @pl.when(s < NS - 1)
        def _():
            base = pl.multiple_of(s * rows_a, 8)
            pltpu.sync_copy(acc_sh.at[pl.ds(base, rows_a)],
                            out_hbm.at[c, pl.ds(base, rows_a)])

        @pl.when(s == NS - 1)
        def _():
            pltpu.sync_copy(acc_sh.at[pl.ds((NS - 1) * rows_a, rows_b)],
                            out_hbm.at[c, pl.ds((NS - 1) * rows_a, rows_b)])

    return spmm


def _matmul(x, W, bm=1000):
    n, f = x.shape
    h = W.shape[1]
    grid = n // bm

    def body(x_ref, w_ref, o_ref):
        o_ref[...] = jnp.dot(x_ref[...], w_ref[...],
                             preferred_element_type=jnp.float32)

    return pl.pallas_call(
        body,
        grid=(grid,),
        in_specs=[pl.BlockSpec((bm, f), lambda i: (i, 0)),
                  pl.BlockSpec((f, h), lambda i: (0, 0))],
        out_specs=pl.BlockSpec((bm, h), lambda i: (i, 0)),
        out_shape=jax.ShapeDtypeStruct((n, h), jnp.float32),
    )(x, W)


def _fuse_matmul(p, b, W, relu, bm=1000):
    _, n, d = p.shape
    h = W.shape[1]
    grid = n // bm
    b2 = b.reshape(1, d)

    def body(p_ref, b_ref, w_ref, o_ref):
        z = p_ref[0] + p_ref[1] + b_ref[...]
        if relu:
            z = jnp.maximum(z, 0.0)
        o_ref[...] = jnp.dot(z, w_ref[...], preferred_element_type=jnp.float32)

    return pl.pallas_call(
        body,
        grid=(grid,),
        in_specs=[pl.BlockSpec((2, bm, d), lambda i: (0, i, 0)),
                  pl.BlockSpec((1, d), lambda i: (0, 0)),
                  pl.BlockSpec((d, h), lambda i: (0, 0))],
        out_specs=pl.BlockSpec((bm, h), lambda i: (i, 0)),
        out_shape=jax.ShapeDtypeStruct((n, h), jnp.float32),
    )(p, b2, W)


def _fuse_logsoftmax(p, b, bm=1000):
    _, n, d = p.shape
    grid = n // bm
    b2 = b.reshape(1, d)

    def body(p_ref, b_ref, o_ref):
        z = p_ref[0] + p_ref[1] + b_ref[...]
        z = z - jnp.max(z, axis=1, keepdims=True)
        o_ref[...] = z - jnp.log(jnp.sum(jnp.exp(z), axis=1, keepdims=True))

    return pl.pallas_call(
        body,
        grid=(grid,),
        in_specs=[pl.BlockSpec((2, bm, d), lambda i: (0, i, 0)),
                  pl.BlockSpec((1, d), lambda i: (0, 0))],
        out_specs=pl.BlockSpec((bm, d), lambda i: (i, 0)),
        out_shape=jax.ShapeDtypeStruct((n, d), jnp.float32),
    )(p, b2)


def kernel(x, edge_index, edge_weight, W1, b1, W2, b2, W3, b3):
    n, nfeat = x.shape
    e = edge_weight.shape[0]
    nhid = W1.shape[1]
    nclass = W3.shape[1]
    src = edge_index[0].reshape(-1, 80)
    dst = edge_index[1].reshape(-1, 80)
    zeros_h = jnp.zeros((n, nhid), jnp.float32)
    zeros_c = jnp.zeros((n, nclass), jnp.float32)

    spmm_h = _make_spmm(n, e, nhid)
    spmm_c = _make_spmm(n, e, nclass)

    s1 = _matmul(x, W1)
    p1 = spmm_h(src, dst, edge_weight, s1, zeros_h)
    s2 = _fuse_matmul(p1, b1, W2, relu=True)
    p2 = spmm_h(src, dst, edge_weight, s2, zeros_h)
    s3 = _fuse_matmul(p2, b2, W3, relu=False)
    p3 = spmm_c(src, dst, edge_weight, s3, zeros_c)
    return _fuse_logsoftmax(p3, b3)


# speedup vs baseline: 12.4156x; 1.2938x over previous
import functools

import jax
import jax.numpy as jnp
from jax import lax
from jax.experimental import pallas as pl
from jax.experimental.pallas import tpu as pltpu
from jax.experimental.pallas import tpu_sc as plsc

NC = 2
NS = 16
L = 16
NW = NC * NS
CHUNK = 40


def _make_spmm(N, E, D, K=CHUNK):
    EW = E // NW
    assert EW * NW == E and EW % K == 0 and K % 8 == 0 and K <= 128
    nchunk = EW // K
    NB = 4
    ngroup = nchunk // NB
    ntail = nchunk - ngroup * NB
    assert ntail >= 2
    rows_a = ((N + NS - 1) // NS + 7) // 8 * 8
    rows_b = N - rows_a * (NS - 1)
    assert rows_b > 0
    mesh = plsc.VectorSubcoreMesh(core_axis_name="c", subcore_axis_name="s")

    @functools.partial(
        pl.kernel,
        out_type=jax.ShapeDtypeStruct((NC, N, D), jnp.float32),
        mesh=mesh,
        compiler_params=pltpu.CompilerParams(needs_layout_passes=False,
                                             use_tc_tiling_on_sc=False),
        scratch_types=[
            pltpu.VMEM((nchunk, K), jnp.int32),
            pltpu.VMEM((nchunk, K), jnp.int32),
            pltpu.VMEM((EW,), jnp.float32),
            [pltpu.VMEM((K, D), jnp.float32) for _ in range(NB)],
            pltpu.VMEM_SHARED((N, D), jnp.float32),
            [pltpu.SemaphoreType.DMA for _ in range(NB)],
            [pltpu.SemaphoreType.DMA for _ in range(NB)],
        ],
    )
    def spmm(src_hbm, dst_hbm, w_hbm, table_hbm, zeros_hbm, out_hbm,
             src_v, dst_v, w_v, rows, acc_sh, gsem, ssem):
        c = lax.axis_index("c")
        s = lax.axis_index("s")
        wid = s * NC + c

        pltpu.sync_copy(src_hbm.at[pl.ds(wid * nchunk, nchunk)], src_v)
        pltpu.sync_copy(dst_hbm.at[pl.ds(wid * nchunk, nchunk)], dst_v)
        pltpu.sync_copy(w_hbm.at[pl.ds(pl.multiple_of(wid * EW, 8), EW)], w_v)

        @pl.when(s < NS - 1)
        def _():
            base = pl.multiple_of(s * rows_a, 8)
            pltpu.sync_copy(zeros_hbm.at[pl.ds(base, rows_a)],
                            acc_sh.at[pl.ds(base, rows_a)])

        @pl.when(s == NS - 1)
        def _():
            pltpu.sync_copy(zeros_hbm.at[pl.ds((NS - 1) * rows_a, rows_b)],
                            acc_sh.at[pl.ds((NS - 1) * rows_a, rows_b)])

        plsc.subcore_barrier()

        def weight(rows_ref, chunk_id):
            @plsc.parallel_loop(0, K, 1, unroll=8)
            def _(e):
                wb = plsc.load_gather(
                    w_v, [jnp.full((L,), chunk_id * K + e, jnp.int32)])
                for j in range(D // L):
                    sl = pl.ds(j * L, L)
                    rows_ref[e, sl] = rows_ref[e, sl] * wb

        def issue_gather(chunk_id, b):
            pltpu.async_copy(table_hbm.at[src_v.at[chunk_id]], rows[b],
                             gsem[b])

        def wait_gather(chunk_id, b):
            pltpu.make_async_copy(table_hbm.at[src_v.at[chunk_id]], rows[b],
                                  gsem[b]).wait()

        def issue_scatter(chunk_id, b):
            pltpu.async_copy(rows[b], acc_sh.at[dst_v.at[chunk_id]],
                             ssem[b], add=True)

        def wait_scatter(chunk_id, b):
            pltpu.make_async_copy(rows[b], acc_sh.at[dst_v.at[chunk_id]],
                                  ssem[b]).wait()

        issue_gather(0, 0)
        issue_gather(1, 1)

        def group(i, carry):
            for b in range(NB):
                g = NB * i + b
                nxt = (b + 2) % NB

                @pl.when(g >= 2)
                def _():
                    wait_scatter(g - 2, nxt)

                @pl.when(g + 2 <= nchunk - 1)
                def _():
                    issue_gather(g + 2, nxt)

                wait_gather(g, b)
                weight(rows[b], g)
                issue_scatter(g, b)
            return carry

        lax.fori_loop(0, ngroup, group, 0)
        for g in range(ngroup * NB, nchunk):
            b = g % NB
            wait_scatter(g - 2, (b + 2) % NB)
            wait_gather(g, b)
            weight(rows[b], g)
            issue_scatter(g, b)
        wait_scatter(nchunk - 2, (nchunk - 2) % NB)
        wait_scatter(nchunk - 1, (nchunk - 1) % NB)
        plsc.subcore_barrier()

        @pl.when(s < NS - 1)
        def _():
            base = pl.multiple_of(s * rows_a, 8)
            pltpu.sync_copy(acc_sh.at[pl.ds(base, rows_a)],
                            out_hbm.at[c, pl.ds(base, rows_a)])

        @pl.when(s == NS - 1)
        def _():
            pltpu.sync_copy(acc_sh.at[pl.ds((NS - 1) * rows_a, rows_b)],
                            out_hbm.at[c, pl.ds((NS - 1) * rows_a, rows_b)])

    return spmm


def _matmul(x, W, bm=1000):
    n, f = x.shape
    h = W.shape[1]
    grid = n // bm

    def body(x_ref, w_ref, o_ref):
        o_ref[...] = jnp.dot(x_ref[...], w_ref[...],
                             preferred_element_type=jnp.float32)

    return pl.pallas_call(
        body,
        grid=(grid,),
        in_specs=[pl.BlockSpec((bm, f), lambda i: (i, 0)),
                  pl.BlockSpec((f, h), lambda i: (0, 0))],
        out_specs=pl.BlockSpec((bm, h), lambda i: (i, 0)),
        out_shape=jax.ShapeDtypeStruct((n, h), jnp.float32),
    )(x, W)


def _fuse_matmul(p, b, W, relu, bm=1000):
    _, n, d = p.shape
    h = W.shape[1]
    grid = n // bm
    b2 = b.reshape(1, d)

    def body(p_ref, b_ref, w_ref, o_ref):
        z = p_ref[0] + p_ref[1] + b_ref[...]
        if relu:
            z = jnp.maximum(z, 0.0)
        o_ref[...] = jnp.dot(z, w_ref[...], preferred_element_type=jnp.float32)

    return pl.pallas_call(
        body,
        grid=(grid,),
        in_specs=[pl.BlockSpec((2, bm, d), lambda i: (0, i, 0)),
                  pl.BlockSpec((1, d), lambda i: (0, 0)),
                  pl.BlockSpec((d, h), lambda i: (0, 0))],
        out_specs=pl.BlockSpec((bm, h), lambda i: (i, 0)),
        out_shape=jax.ShapeDtypeStruct((n, h), jnp.float32),
    )(p, b2, W)


def _fuse_logsoftmax(p, b, bm=1000):
    _, n, d = p.shape
    grid = n // bm
    b2 = b.reshape(1, d)

    def body(p_ref, b_ref, o_ref):
        z = p_ref[0] + p_ref[1] + b_ref[...]
        z = z - jnp.max(z, axis=1, keepdims=True)
        o_ref[...] = z - jnp.log(jnp.sum(jnp.exp(z), axis=1, keepdims=True))

    return pl.pallas_call(
        body,
        grid=(grid,),
        in_specs=[pl.BlockSpec((2, bm, d), lambda i: (0, i, 0)),
                  pl.BlockSpec((1, d), lambda i: (0, 0))],
        out_specs=pl.BlockSpec((bm, d), lambda i: (i, 0)),
        out_shape=jax.ShapeDtypeStruct((n, d), jnp.float32),
    )(p, b2)


def kernel(x, edge_index, edge_weight, W1, b1, W2, b2, W3, b3):
    n, nfeat = x.shape
    e = edge_weight.shape[0]
    nhid = W1.shape[1]
    nclass = W3.shape[1]
    src = edge_index[0].reshape(-1, CHUNK)
    dst = edge_index[1].reshape(-1, CHUNK)
    zeros_h = jnp.zeros((n, nhid), jnp.float32)
    zeros_c = jnp.zeros((n, nclass), jnp.float32)

    spmm_h = _make_spmm(n, e, nhid)
    spmm_c = _make_spmm(n, e, nclass)

    s1 = _matmul(x, W1)
    p1 = spmm_h(src, dst, edge_weight, s1, zeros_h)
    s2 = _fuse_matmul(p1, b1, W2, relu=True)
    p2 = spmm_h(src, dst, edge_weight, s2, zeros_h)
    s3 = _fuse_matmul(p2, b2, W3, relu=False)
    p3 = spmm_c(src, dst, edge_weight, s3, zeros_c)
    return _fuse_logsoftmax(p3, b3)
